# Initial kernel scaffold; baseline (speedup 1.0000x reference)
import jax
import jax.numpy as jnp
from jax.experimental import pallas as pl


def kernel(x, router_wq, router_wk, Wq, bq, Wk, bk, Wv, bv, Wo, bo, W1, b1, W2, b2, ln1_w, ln1_b, ln2_w, ln2_b):
    raise NotImplementedError("write your pallas kernel here")


# speedup vs baseline: 2.1734x; 2.1734x over previous
import functools
import math

import jax
import jax.numpy as jnp
from jax import lax
from jax.experimental import pallas as pl
from jax.experimental.pallas import tpu as pltpu
from jax.experimental.pallas import tpu_sc as plsc

B = 2
N = 4096
D = 1024
H = 16
HD = 64
L = 4
KK = 409
KP = 512
NP = 4224
R = B * NP
TILE = 256
NT = R // TILE
GK = B * KP
RANK = 16
NW = 32
BPW = GK // NW
F32 = jnp.float32
BF16 = jnp.bfloat16
INT_MIN = -2147483648


def _dot_t(a, b, precision=None):
    return lax.dot_general(a, b, (((1,), (1,)), ((), ())),
                           preferred_element_type=F32, precision=precision)


def _cumsum2d(m):
    i0 = lax.broadcasted_iota(jnp.int32, (128, 128), 0)
    i1 = lax.broadcasted_iota(jnp.int32, (128, 128), 1)
    u = (i0 <= i1).astype(BF16)
    cin = jnp.dot(m.astype(BF16), u, preferred_element_type=F32)
    t = cin[:, 127:128]
    r0 = lax.broadcasted_iota(jnp.int32, (32, 32), 0)
    r1 = lax.broadcasted_iota(jnp.int32, (32, 32), 1)
    ls = (r0 > r1).astype(BF16)
    off = jnp.dot(ls, t.astype(BF16), preferred_element_type=F32)
    return cin + off


def _router_body(x_ref, rwq_ref, rwk_ref, idx_ref):
    b = pl.program_id(0)
    xb2 = x_ref[...].astype(BF16)
    if True:
        qb = _dot_t(xb2, rwq_ref[...].astype(BF16))
        kb = _dot_t(xb2, rwk_ref[...].astype(BF16))
        qg = jnp.sum(qb, axis=0, keepdims=True) * (1.0 / N)
        srow = _dot_t(qg.astype(BF16), kb.astype(BF16))
        s2d = jnp.concatenate(
            [srow[:, i * 128:(i + 1) * 128] for i in range(32)], axis=0)
        bits = lax.bitcast_convert_type(s2d, jnp.int32)
        skey = jnp.where(bits >= 0, bits, bits ^ jnp.int32(0x7FFFFFFF))

        def step(i, tu):
            cand = tu | lax.shift_left(jnp.int32(1), 31 - i)
            thr = cand ^ jnp.int32(INT_MIN)
            cnt = jnp.sum((skey >= thr).astype(jnp.int32))
            return jnp.where(cnt >= KK, cand, tu)

        tu = lax.fori_loop(0, 32, step, jnp.int32(0))
        thr = tu ^ jnp.int32(INT_MIN)
        gt = skey > thr
        eqm = skey == thr
        m = jnp.sum(gt.astype(jnp.int32))
        cum_eq = _cumsum2d(eqm.astype(F32))
        sel = gt | (eqm & (cum_eq <= (KK - m).astype(F32)))
        c = _cumsum2d(sel.astype(F32))
        ci = c.astype(jnp.int32)
        cm = jnp.where(sel, ci - 1, -1)
        nmat = (128 * lax.broadcasted_iota(jnp.int32, (32, 128), 0)
                + lax.broadcasted_iota(jnp.int32, (32, 128), 1))
        cm3 = lax.broadcast_in_dim(cm, (32, 128, 128), (0, 1))
        nm3 = lax.broadcast_in_dim(nmat, (32, 128, 128), (0, 1))
        qio = lax.broadcasted_iota(jnp.int32, (32, 128, 128), 2)
        blocks = []
        for a in range(KP // 128):
            hit = (cm3 - a * 128) == qio
            v = jnp.sum(jnp.where(hit, nm3, 0), axis=0)
            blocks.append(jnp.sum(v, axis=0, keepdims=True))
        idxrow = jnp.concatenate(blocks, axis=1)
        pv = lax.broadcasted_iota(jnp.int32, (1, KP), 1)
        idxrow = jnp.where(pv < KK, idxrow + b * NP, b * NP + N)
        idx_ref[0] = idxrow


def _router(x2, rwq, rwk):
    out = pl.pallas_call(
        _router_body,
        grid=(B,),
        in_specs=[
            pl.BlockSpec((N, D), lambda b: (b, 0)),
            pl.BlockSpec((RANK, D), lambda b: (0, 0)),
            pl.BlockSpec((RANK, D), lambda b: (0, 0)),
        ],
        out_specs=pl.BlockSpec((1, 1, KP), lambda b: (b, 0, 0)),
        out_shape=jax.ShapeDtypeStruct((B, 1, KP), jnp.int32),
    )(x2, rwq, rwk)
    return out.reshape(B, KP)


def _ln_body(x_ref, w_ref, b_ref, o_ref):
    x = x_ref[...]
    mu = jnp.mean(x, axis=1, keepdims=True)
    xc = x - mu
    var = jnp.mean(xc * xc, axis=1, keepdims=True)
    o_ref[...] = xc * lax.rsqrt(var + 1e-5) * w_ref[...] + b_ref[...]


def _ln(xr, w, b):
    return pl.pallas_call(
        _ln_body,
        grid=(NT,),
        in_specs=[
            pl.BlockSpec((TILE, D), lambda t: (t, 0)),
            pl.BlockSpec((1, D), lambda t: (0, 0)),
            pl.BlockSpec((1, D), lambda t: (0, 0)),
        ],
        out_specs=pl.BlockSpec((TILE, D), lambda t: (t, 0)),
        out_shape=jax.ShapeDtypeStruct((R, D), F32),
    )(xr, w, b)


def _sc_gather(table, idx):
    mesh = plsc.VectorSubcoreMesh(core_axis_name="c", subcore_axis_name="s")

    @functools.partial(
        pl.kernel, mesh=mesh,
        out_type=jax.ShapeDtypeStruct((GK, D), F32),
        scratch_types=[
            pltpu.VMEM((BPW,), jnp.int32),
            pltpu.VMEM((BPW, D), F32),
            pltpu.SemaphoreType.DMA,
        ],
    )
    def k(table_hbm, idx_hbm, out_hbm, idx_v, rows_v, sem):
        wid = lax.axis_index("s") * 2 + lax.axis_index("c")
        base = wid * BPW
        pltpu.sync_copy(idx_hbm.at[pl.ds(base, BPW)], idx_v)
        pltpu.async_copy(table_hbm.at[idx_v], rows_v, sem).wait()
        pltpu.sync_copy(rows_v, out_hbm.at[pl.ds(base, BPW)])

    return k(table, idx)


def _attn_body(xs_ref, wq_ref, wk_ref, wv_ref, wo_ref,
               bq_ref, bk_ref, bv_ref, bo_ref, o_ref):
    xs = xs_ref[0]
    xb = xs.astype(BF16)
    q = _dot_t(xb, wq_ref[...]) + bq_ref[...]
    k = _dot_t(xb, wk_ref[...]) + bk_ref[...]
    v = _dot_t(xb, wv_ref[...]) + bv_ref[...]
    jmask = jnp.where(
        lax.broadcasted_iota(jnp.int32, (1, KP), 1) < KK, 0.0, -1e30)
    heads = []
    scale = 1.0 / math.sqrt(HD)
    for h in range(H):
        sl = slice(h * HD, (h + 1) * HD)
        qh = q[:, sl].astype(BF16)
        kh = k[:, sl].astype(BF16)
        vh = v[:, sl].astype(BF16)
        s = _dot_t(qh, kh) * scale + jmask
        mx = jnp.max(s, axis=1, keepdims=True)
        e = jnp.exp(s - mx)
        p = e / jnp.sum(e, axis=1, keepdims=True)
        heads.append(jnp.dot(p.astype(BF16), vh, preferred_element_type=F32))
    cat = jnp.concatenate(heads, axis=1).astype(BF16)
    attn = _dot_t(cat, wo_ref[...]) + bo_ref[...]
    o_ref[0] = (attn - xs).astype(BF16)


def _attn(xs3, wq, wk, wv, wo, bq, bk, bv, bo):
    wspec = pl.BlockSpec((D, D), lambda b: (0, 0))
    bspec = pl.BlockSpec((1, D), lambda b: (0, 0))
    return pl.pallas_call(
        _attn_body,
        grid=(B,),
        in_specs=[pl.BlockSpec((1, KP, D), lambda b: (b, 0, 0)),
                  wspec, wspec, wspec, wspec,
                  bspec, bspec, bspec, bspec],
        out_specs=pl.BlockSpec((1, KP, D), lambda b: (b, 0, 0)),
        out_shape=jax.ShapeDtypeStruct((B, KP, D), BF16),
    )(xs3, wq, wk, wv, wo, bq, bk, bv, bo)


def _erf(z):
    s = jnp.sign(z)
    za = jnp.abs(z)
    t = 1.0 / (1.0 + 0.3275911 * za)
    poly = t * (0.254829592 + t * (-0.284496736 + t * (1.421413741
                + t * (-1.453152027 + t * 1.061405429))))
    return s * (1.0 - poly * jnp.exp(-za * za))


def _ffn_body(x_ref, nx_ref, idx_ref, delta_ref, w1_ref, w2_ref,
              lw_ref, lb_ref, b1_ref, b2_ref, o_ref):
    t = pl.program_id(0)
    xt = x_ref[...]
    rowid = (t * TILE
             + lax.broadcasted_iota(jnp.int32, (TILE, 1), 0))
    eq = (idx_ref[...] == rowid).astype(BF16)
    patch = jnp.dot(eq, delta_ref[...].astype(BF16),
                    preferred_element_type=F32)
    x1 = xt + nx_ref[...] + patch
    mu = jnp.mean(x1, axis=1, keepdims=True)
    xc = x1 - mu
    var = jnp.mean(xc * xc, axis=1, keepdims=True)
    ln = xc * lax.rsqrt(var + 1e-5) * lw_ref[...] + lb_ref[...]
    h = _dot_t(ln.astype(BF16), w1_ref[...]) + b1_ref[...]
    g = 0.5 * h * (1.0 + _erf(h * 0.7071067811865476))
    y = _dot_t(g.astype(BF16), w2_ref[...]) + b2_ref[...]
    o_ref[...] = x1 + y


def _ffn(xr, nx, idx_row, delta2, w1, w2, lw, lb, b1, b2):
    rspec = pl.BlockSpec((TILE, D), lambda t: (t, 0))
    cspec = pl.BlockSpec((1, D), lambda t: (0, 0))
    return pl.pallas_call(
        _ffn_body,
        grid=(NT,),
        in_specs=[
            rspec, rspec,
            pl.BlockSpec((1, GK), lambda t: (0, 0)),
            pl.BlockSpec((GK, D), lambda t: (0, 0)),
            pl.BlockSpec((4 * D, D), lambda t: (0, 0)),
            pl.BlockSpec((D, 4 * D), lambda t: (0, 0)),
            cspec, cspec,
            pl.BlockSpec((1, 4 * D), lambda t: (0, 0)),
            cspec,
        ],
        out_specs=rspec,
        out_shape=jax.ShapeDtypeStruct((R, D), F32),
    )(xr, nx, idx_row, delta2, w1, w2, lw, lb, b1, b2)


def kernel(x, router_wq, router_wk, Wq, bq, Wk, bk, Wv, bv, Wo, bo,
           W1, b1, W2, b2, ln1_w, ln1_b, ln2_w, ln2_b):
    x2 = x.reshape(B * N, D)
    idx2 = _router(x2, router_wq, router_wk)
    idx_flat = idx2.reshape(GK)
    idx_row = idx2.reshape(1, GK)
    xr = jnp.pad(x, ((0, 0), (0, NP - N), (0, 0))).reshape(R, D)
    Wq_b, Wk_b, Wv_b, Wo_b = (w.astype(BF16) for w in (Wq, Wk, Wv, Wo))
    W1_b, W2_b = W1.astype(BF16), W2.astype(BF16)
    for i in range(L):
        nx = _ln(xr, ln1_w[i].reshape(1, D), ln1_b[i].reshape(1, D))
        xs = _sc_gather(nx, idx_flat)
        delta = _attn(xs.reshape(B, KP, D),
                      Wq_b[i], Wk_b[i], Wv_b[i], Wo_b[i],
                      bq[i].reshape(1, D), bk[i].reshape(1, D),
                      bv[i].reshape(1, D), bo[i].reshape(1, D))
        xr = _ffn(xr, nx, idx_row, delta.reshape(GK, D),
                  W1_b[i], W2_b[i],
                  ln2_w[i].reshape(1, D), ln2_b[i].reshape(1, D),
                  b1[i].reshape(1, 4 * D), b2[i].reshape(1, D))
    return xr.reshape(B, NP, D)[:, :N, :]


# speedup vs baseline: 2.2533x; 1.0368x over previous
import functools
import math

import jax
import jax.numpy as jnp
from jax import lax
from jax.experimental import pallas as pl
from jax.experimental.pallas import tpu as pltpu
from jax.experimental.pallas import tpu_sc as plsc

B = 2
N = 4096
D = 1024
H = 16
HD = 64
L = 4
KK = 409
KP = 512
NP = 4224
R = B * NP
TILE = 256
NT = R // TILE
GK = B * KP
RANK = 16
NW = 32
BPW = GK // NW
F32 = jnp.float32
BF16 = jnp.bfloat16
INT_MIN = -2147483648


def _dot_t(a, b, precision=None):
    return lax.dot_general(a, b, (((1,), (1,)), ((), ())),
                           preferred_element_type=F32, precision=precision)


def _cumsum2d(m):
    i0 = lax.broadcasted_iota(jnp.int32, (128, 128), 0)
    i1 = lax.broadcasted_iota(jnp.int32, (128, 128), 1)
    u = (i0 <= i1).astype(BF16)
    cin = jnp.dot(m.astype(BF16), u, preferred_element_type=F32)
    t = cin[:, 127:128]
    r0 = lax.broadcasted_iota(jnp.int32, (32, 32), 0)
    r1 = lax.broadcasted_iota(jnp.int32, (32, 32), 1)
    ls = (r0 > r1).astype(BF16)
    off = jnp.dot(ls, t.astype(BF16), preferred_element_type=F32)
    return cin + off


def _router_body(x_ref, rwq_ref, rwk_ref, idx_ref):
    b = pl.program_id(0)
    xb2 = x_ref[...].astype(BF16)
    if True:
        qb = _dot_t(xb2, rwq_ref[...].astype(BF16))
        kb = _dot_t(xb2, rwk_ref[...].astype(BF16))
        qg = jnp.sum(qb, axis=0, keepdims=True) * (1.0 / N)
        srow = _dot_t(qg.astype(BF16), kb.astype(BF16))
        s2d = jnp.concatenate(
            [srow[:, i * 128:(i + 1) * 128] for i in range(32)], axis=0)
        bits = lax.bitcast_convert_type(s2d, jnp.int32)
        skey = jnp.where(bits >= 0, bits, bits ^ jnp.int32(0x7FFFFFFF))

        def step(i, tu):
            cand = tu | lax.shift_left(jnp.int32(1), 31 - i)
            thr = cand ^ jnp.int32(INT_MIN)
            cnt = jnp.sum((skey >= thr).astype(jnp.int32))
            return jnp.where(cnt >= KK, cand, tu)

        tu = lax.fori_loop(0, 32, step, jnp.int32(0))
        thr = tu ^ jnp.int32(INT_MIN)
        gt = skey > thr
        eqm = skey == thr
        m = jnp.sum(gt.astype(jnp.int32))
        cum_eq = _cumsum2d(eqm.astype(F32))
        sel = gt | (eqm & (cum_eq <= (KK - m).astype(F32)))
        c = _cumsum2d(sel.astype(F32))
        ci = c.astype(jnp.int32)
        cm = jnp.where(sel, ci - 1, -1)
        nmat = (128 * lax.broadcasted_iota(jnp.int32, (32, 128), 0)
                + lax.broadcasted_iota(jnp.int32, (32, 128), 1))
        cm3 = lax.broadcast_in_dim(cm, (32, 128, 128), (0, 1))
        nm3 = lax.broadcast_in_dim(nmat, (32, 128, 128), (0, 1))
        qio = lax.broadcasted_iota(jnp.int32, (32, 128, 128), 2)
        blocks = []
        for a in range(KP // 128):
            hit = (cm3 - a * 128) == qio
            v = jnp.sum(jnp.where(hit, nm3, 0), axis=0)
            blocks.append(jnp.sum(v, axis=0, keepdims=True))
        idxrow = jnp.concatenate(blocks, axis=1)
        pv = lax.broadcasted_iota(jnp.int32, (1, KP), 1)
        idxrow = jnp.where(pv < KK, idxrow + b * NP, b * NP + N)
        idx_ref[0] = idxrow


def _router(x2, rwq, rwk):
    out = pl.pallas_call(
        _router_body,
        grid=(B,),
        in_specs=[
            pl.BlockSpec((N, D), lambda b: (b, 0)),
            pl.BlockSpec((RANK, D), lambda b: (0, 0)),
            pl.BlockSpec((RANK, D), lambda b: (0, 0)),
        ],
        out_specs=pl.BlockSpec((1, 1, KP), lambda b: (b, 0, 0)),
        out_shape=jax.ShapeDtypeStruct((B, 1, KP), jnp.int32),
    )(x2, rwq, rwk)
    return out.reshape(B, KP)


def _ln_body(x_ref, w_ref, b_ref, o_ref):
    x = x_ref[...]
    mu = jnp.mean(x, axis=1, keepdims=True)
    xc = x - mu
    var = jnp.mean(xc * xc, axis=1, keepdims=True)
    o_ref[...] = xc * lax.rsqrt(var + 1e-5) * w_ref[...] + b_ref[...]


def _ln(xr, w, b):
    return pl.pallas_call(
        _ln_body,
        grid=(NT,),
        in_specs=[
            pl.BlockSpec((TILE, D), lambda t: (t, 0)),
            pl.BlockSpec((1, D), lambda t: (0, 0)),
            pl.BlockSpec((1, D), lambda t: (0, 0)),
        ],
        out_specs=pl.BlockSpec((TILE, D), lambda t: (t, 0)),
        out_shape=jax.ShapeDtypeStruct((R, D), F32),
    )(xr, w, b)


def _sc_gather(table, idx):
    mesh = plsc.VectorSubcoreMesh(core_axis_name="c", subcore_axis_name="s")

    @functools.partial(
        pl.kernel, mesh=mesh,
        out_type=jax.ShapeDtypeStruct((GK, D), F32),
        scratch_types=[
            pltpu.VMEM((BPW,), jnp.int32),
            pltpu.VMEM((BPW, D), F32),
            pltpu.SemaphoreType.DMA,
        ],
    )
    def k(table_hbm, idx_hbm, out_hbm, idx_v, rows_v, sem):
        wid = lax.axis_index("s") * 2 + lax.axis_index("c")
        base = wid * BPW
        pltpu.sync_copy(idx_hbm.at[pl.ds(base, BPW)], idx_v)
        pltpu.async_copy(table_hbm.at[idx_v], rows_v, sem).wait()
        pltpu.sync_copy(rows_v, out_hbm.at[pl.ds(base, BPW)])

    return k(table, idx)


def _attn_body(xs_ref, wq_ref, wk_ref, wv_ref, wo_ref,
               bq_ref, bk_ref, bv_ref, bo_ref, o_ref):
    xs = xs_ref[0]
    xb = xs.astype(BF16)
    q = _dot_t(xb, wq_ref[...]) + bq_ref[...]
    k = _dot_t(xb, wk_ref[...]) + bk_ref[...]
    v = _dot_t(xb, wv_ref[...]) + bv_ref[...]
    jmask = jnp.where(
        lax.broadcasted_iota(jnp.int32, (1, KP), 1) < KK, 0.0, -1e30)
    heads = []
    scale = 1.0 / math.sqrt(HD)
    for h in range(H):
        sl = slice(h * HD, (h + 1) * HD)
        qh = q[:, sl].astype(BF16)
        kh = k[:, sl].astype(BF16)
        vh = v[:, sl].astype(BF16)
        s = _dot_t(qh, kh) * scale + jmask
        mx = jnp.max(s, axis=1, keepdims=True)
        e = jnp.exp(s - mx)
        p = e / jnp.sum(e, axis=1, keepdims=True)
        heads.append(jnp.dot(p.astype(BF16), vh, preferred_element_type=F32))
    cat = jnp.concatenate(heads, axis=1).astype(BF16)
    attn = _dot_t(cat, wo_ref[...]) + bo_ref[...]
    o_ref[0] = (attn - xs).astype(BF16)


def _attn(xs3, wq, wk, wv, wo, bq, bk, bv, bo):
    wspec = pl.BlockSpec((D, D), lambda b: (0, 0))
    bspec = pl.BlockSpec((1, D), lambda b: (0, 0))
    return pl.pallas_call(
        _attn_body,
        grid=(B,),
        in_specs=[pl.BlockSpec((1, KP, D), lambda b: (b, 0, 0)),
                  wspec, wspec, wspec, wspec,
                  bspec, bspec, bspec, bspec],
        out_specs=pl.BlockSpec((1, KP, D), lambda b: (b, 0, 0)),
        out_shape=jax.ShapeDtypeStruct((B, KP, D), BF16),
    )(xs3, wq, wk, wv, wo, bq, bk, bv, bo)


def _erf(z):
    s = jnp.sign(z)
    za = jnp.abs(z)
    t = 1.0 / (1.0 + 0.3275911 * za)
    poly = t * (0.254829592 + t * (-0.284496736 + t * (1.421413741
                + t * (-1.453152027 + t * 1.061405429))))
    return s * (1.0 - poly * jnp.exp(-za * za))


def _ffn_body(x_ref, nx_ref, idx_ref, delta_ref, w1_ref, w2_ref,
              lw_ref, lb_ref, b1_ref, b2_ref, nw_ref, nb_ref,
              o_ref, n_ref):
    t = pl.program_id(0)
    xt = x_ref[...]
    rowid = (t * TILE
             + lax.broadcasted_iota(jnp.int32, (TILE, 1), 0))
    eq = (idx_ref[...] == rowid).astype(BF16)
    patch = jnp.dot(eq, delta_ref[...].astype(BF16),
                    preferred_element_type=F32)
    x1 = xt + nx_ref[...] + patch
    mu = jnp.mean(x1, axis=1, keepdims=True)
    xc = x1 - mu
    var = jnp.mean(xc * xc, axis=1, keepdims=True)
    ln = xc * lax.rsqrt(var + 1e-5) * lw_ref[...] + lb_ref[...]
    h = _dot_t(ln.astype(BF16), w1_ref[...]) + b1_ref[...]
    g = 0.5 * h * (1.0 + _erf(h * 0.7071067811865476))
    y = _dot_t(g.astype(BF16), w2_ref[...]) + b2_ref[...]
    x2 = x1 + y
    o_ref[...] = x2
    mu2 = jnp.mean(x2, axis=1, keepdims=True)
    xc2 = x2 - mu2
    var2 = jnp.mean(xc2 * xc2, axis=1, keepdims=True)
    nx_ref2 = xc2 * lax.rsqrt(var2 + 1e-5) * nw_ref[...] + nb_ref[...]
    n_ref[...] = nx_ref2


def _ffn(xr, nx, idx_row, delta2, w1, w2, lw, lb, b1, b2, nw, nb):
    rspec = pl.BlockSpec((TILE, D), lambda t: (t, 0))
    cspec = pl.BlockSpec((1, D), lambda t: (0, 0))
    return pl.pallas_call(
        _ffn_body,
        grid=(NT,),
        in_specs=[
            rspec, rspec,
            pl.BlockSpec((1, GK), lambda t: (0, 0)),
            pl.BlockSpec((GK, D), lambda t: (0, 0)),
            pl.BlockSpec((4 * D, D), lambda t: (0, 0)),
            pl.BlockSpec((D, 4 * D), lambda t: (0, 0)),
            cspec, cspec,
            pl.BlockSpec((1, 4 * D), lambda t: (0, 0)),
            cspec, cspec, cspec,
        ],
        out_specs=[rspec, rspec],
        out_shape=[jax.ShapeDtypeStruct((R, D), F32),
                   jax.ShapeDtypeStruct((R, D), F32)],
    )(xr, nx, idx_row, delta2, w1, w2, lw, lb, b1, b2, nw, nb)


def kernel(x, router_wq, router_wk, Wq, bq, Wk, bk, Wv, bv, Wo, bo,
           W1, b1, W2, b2, ln1_w, ln1_b, ln2_w, ln2_b):
    x2 = x.reshape(B * N, D)
    idx2 = _router(x2, router_wq, router_wk)
    idx_flat = idx2.reshape(GK)
    idx_row = idx2.reshape(1, GK)
    xr = jnp.pad(x, ((0, 0), (0, NP - N), (0, 0))).reshape(R, D)
    Wq_b, Wk_b, Wv_b, Wo_b = (w.astype(BF16) for w in (Wq, Wk, Wv, Wo))
    W1_b, W2_b = W1.astype(BF16), W2.astype(BF16)
    nx = _ln(xr, ln1_w[0].reshape(1, D), ln1_b[0].reshape(1, D))
    for i in range(L):
        xs = _sc_gather(nx, idx_flat)
        delta = _attn(xs.reshape(B, KP, D),
                      Wq_b[i], Wk_b[i], Wv_b[i], Wo_b[i],
                      bq[i].reshape(1, D), bk[i].reshape(1, D),
                      bv[i].reshape(1, D), bo[i].reshape(1, D))
        j = min(i + 1, L - 1)
        xr, nx = _ffn(xr, nx, idx_row, delta.reshape(GK, D),
                      W1_b[i], W2_b[i],
                      ln2_w[i].reshape(1, D), ln2_b[i].reshape(1, D),
                      b1[i].reshape(1, 4 * D), b2[i].reshape(1, D),
                      ln1_w[j].reshape(1, D), ln1_b[j].reshape(1, D))
    return xr.reshape(B, NP, D)[:, :N, :]
